# baseline (speedup 1.0000x reference)
import functools

import jax
import jax.numpy as jnp
from jax import lax
from jax.experimental import pallas as pl
from jax.experimental.pallas import tpu as pltpu
from jax.experimental.pallas import tpu_sc as plsc

_LANES = 16
_NC = 2
_NS = 16
_NW = _NC * _NS


def _sc_body(n_total, n_pos, chunk, x_hbm, p_hbm, o_hbm, x_v, p_v):
    per_w = n_total // _NW
    n_chunks = per_w // chunk
    wid = lax.axis_index("s") * _NC + lax.axis_index("c")
    base_w = wid * per_w
    pbase_w = (base_w % n_pos).astype(jnp.int32)

    def chunk_body(k, _):
        base = base_w + k * chunk
        pbase = pbase_w + k * chunk
        pltpu.sync_copy(x_hbm.at[pl.ds(base, chunk)], x_v)
        pltpu.sync_copy(p_hbm.at[pl.ds(pbase, chunk)], p_v)

        @plsc.parallel_loop(0, chunk, _LANES, unroll=8)
        def _(i):
            x_v[pl.ds(i, _LANES)] = x_v[pl.ds(i, _LANES)] + p_v[pl.ds(i, _LANES)]

        pltpu.sync_copy(x_v, o_hbm.at[pl.ds(base, chunk)])
        return 0

    lax.fori_loop(0, n_chunks, chunk_body, 0)


def _sc_add(xf, pf, chunk=32 * 1024):
    n_total = xf.shape[0]
    n_pos = pf.shape[0]
    body = functools.partial(_sc_body, n_total, n_pos, chunk)
    mesh = plsc.VectorSubcoreMesh(core_axis_name="c", subcore_axis_name="s")
    return pl.kernel(
        body,
        out_type=jax.ShapeDtypeStruct((n_total,), jnp.float32),
        mesh=mesh,
        scratch_types=[
            pltpu.VMEM((chunk,), jnp.float32),
            pltpu.VMEM((chunk,), jnp.float32),
        ],
    )(xf, pf)


def kernel(x, pos_table):
    B, S, D = x.shape
    pos = pos_table[:S]
    out = _sc_add(x.reshape(-1), pos.reshape(-1))
    return out.reshape(B, S, D)


# speedup vs baseline: 5.5826x; 5.5826x over previous
import jax
import jax.numpy as jnp
from jax.experimental import pallas as pl
from jax.experimental.pallas import tpu as pltpu

_BLOCK_R = 1024


def _add_kernel(x_ref, p_ref, o_ref):
    i = pl.program_id(0)
    n_pos_blocks = p_ref.shape[0] // x_ref.shape[0]
    j = jax.lax.rem(i, n_pos_blocks)
    o_ref[...] = x_ref[...] + p_ref[pl.ds(j * x_ref.shape[0], x_ref.shape[0]), :]


def kernel(x, pos_table):
    B, S, D = x.shape
    pos = pos_table[:S]
    xf = x.reshape(B * S, D)
    grid = (B * S // _BLOCK_R,)
    out = pl.pallas_call(
        _add_kernel,
        grid=grid,
        in_specs=[
            pl.BlockSpec((_BLOCK_R, D), lambda i: (i, 0)),
            pl.BlockSpec((S, D), lambda i: (0, 0)),
        ],
        out_specs=pl.BlockSpec((_BLOCK_R, D), lambda i: (i, 0)),
        out_shape=jax.ShapeDtypeStruct((B * S, D), x.dtype),
        compiler_params=pltpu.CompilerParams(
            dimension_semantics=("arbitrary",),
        ),
    )(xf, pos)
    return out.reshape(B, S, D)
